# Initial kernel scaffold; baseline (speedup 1.0000x reference)
import jax
import jax.numpy as jnp
from jax.experimental import pallas as pl


def kernel(hidden_states, W_router, w_gate, w_up, w_down, ws_gate, ws_up, ws_down):
    raise NotImplementedError("write your pallas kernel here")


# speedup vs baseline: 2.1814x; 2.1814x over previous
import functools

import jax
import jax.numpy as jnp
from jax import lax
from jax.experimental import pallas as pl
from jax.experimental.pallas import tpu as pltpu
from jax.experimental.pallas import tpu_sc as plsc

E = 8
D = 1024
FF = 2048
T = 2048
BT = 128
NTB = T // BT
BLK = 128
P = 3072
NB = P // BLK
LANES = 128

SC_NC = 2
SC_NS = 16
SC_NW = SC_NC * SC_NS
TPW = T // SC_NW


def _route_body(x_ref, wr_ref, pos_ref, wb_ref, beb_ref,
                rank_s, eid_s, wgt_s, cnt_s):
    i = pl.program_id(0)

    @pl.when(i < NTB)
    def _phase1():
        @pl.when(i == 0)
        def _init():
            cnt_s[...] = jnp.zeros_like(cnt_s)

        x = x_ref[...]
        wr = wr_ref[...]
        logits = jnp.dot(x, wr, preferred_element_type=jnp.float32)
        lane = lax.broadcasted_iota(jnp.int32, (BT, LANES), 1)
        valid = lane < E
        neg = jnp.where(valid, logits, -1e30)
        m = jnp.max(neg, axis=1, keepdims=True)
        eid = jnp.min(jnp.where((neg == m) & valid, lane, LANES),
                      axis=1, keepdims=True)
        oh = (lane == eid).astype(jnp.float32)
        r_io = lax.broadcasted_iota(jnp.int32, (BT, BT), 0)
        c_io = lax.broadcasted_iota(jnp.int32, (BT, BT), 1)
        tri = (r_io > c_io).astype(jnp.float32)
        rank_loc = jnp.dot(tri, oh, preferred_element_type=jnp.float32)
        rank = rank_loc + cnt_s[...]
        w = jax.nn.sigmoid(m)
        rank_s[pl.ds(i * BT, BT), :] = jnp.sum(rank * oh, axis=1, keepdims=True)
        eid_s[pl.ds(i * BT, BT), :] = eid
        wgt_s[pl.ds(i * BT, BT), :] = w
        cnt_s[...] = cnt_s[...] + jnp.sum(oh, axis=0, keepdims=True)

    @pl.when(i >= NTB)
    def _phase2():
        j = i - NTB
        cnt = cnt_s[...]
        cnt_pad = jnp.ceil(cnt / BLK) * BLK
        r_io = lax.broadcasted_iota(jnp.int32, (LANES, LANES), 0)
        c_io = lax.broadcasted_iota(jnp.int32, (LANES, LANES), 1)
        tri_u = (r_io < c_io).astype(jnp.float32)
        pad_off = jnp.dot(cnt_pad, tri_u, preferred_element_type=jnp.float32)
        pad_end = pad_off + cnt_pad

        eid = eid_s[pl.ds(j * BT, BT), :]
        rank = rank_s[pl.ds(j * BT, BT), :]
        lane = lax.broadcasted_iota(jnp.int32, (BT, LANES), 1)
        oh = (lane == eid).astype(jnp.float32)
        off_tok = jnp.sum(oh * pad_off, axis=1, keepdims=True)
        pos = (off_tok + rank).astype(jnp.int32)
        pos_ref[...] = jnp.broadcast_to(pos, (BT, LANES))
        wb_ref[...] = jnp.broadcast_to(wgt_s[pl.ds(j * BT, BT), :], (BT, LANES))

        @pl.when(j == 0)
        def _meta():
            blk_start = (lax.broadcasted_iota(jnp.int32, (LANES, LANES), 0)
                         * BLK).astype(jnp.float32)
            lane2 = lax.broadcasted_iota(jnp.int32, (LANES, LANES), 1)
            endb = jnp.broadcast_to(pad_end, (LANES, LANES))
            done = ((blk_start >= endb) & (lane2 < E)).astype(jnp.float32)
            be = jnp.sum(done, axis=1, keepdims=True)
            beb_ref[...] = jnp.broadcast_to(be.astype(jnp.int32),
                                            (LANES, LANES))


def _route_call(hidden, wr_pad):
    return pl.pallas_call(
        _route_body,
        grid=(2 * NTB,),
        in_specs=[
            pl.BlockSpec((BT, D), lambda i: (jnp.where(i < NTB, i, 0), 0)),
            pl.BlockSpec((D, LANES), lambda i: (0, 0)),
        ],
        out_specs=[
            pl.BlockSpec((BT, LANES),
                         lambda i: (jnp.where(i < NTB, 0, i - NTB), 0)),
            pl.BlockSpec((BT, LANES),
                         lambda i: (jnp.where(i < NTB, 0, i - NTB), 0)),
            pl.BlockSpec((LANES, LANES), lambda i: (0, 0)),
        ],
        out_shape=[
            jax.ShapeDtypeStruct((T, LANES), jnp.int32),
            jax.ShapeDtypeStruct((T, LANES), jnp.float32),
            jax.ShapeDtypeStruct((LANES, LANES), jnp.int32),
        ],
        scratch_shapes=[
            pltpu.VMEM((T, 1), jnp.float32),
            pltpu.VMEM((T, 1), jnp.int32),
            pltpu.VMEM((T, 1), jnp.float32),
            pltpu.VMEM((1, LANES), jnp.float32),
        ],
    )(hidden, wr_pad)


def _sc_dispatch(hidden, pos):
    mesh = plsc.VectorSubcoreMesh(core_axis_name="c", subcore_axis_name="s",
                                  num_cores=SC_NC, num_subcores=SC_NS)

    @functools.partial(
        pl.kernel,
        out_type=jax.ShapeDtypeStruct((P, D), jnp.float32),
        mesh=mesh,
        scratch_types=[
            pltpu.VMEM((TPW,), jnp.int32),
            pltpu.VMEM((TPW, D), jnp.float32),
            pltpu.SemaphoreType.DMA,
        ],
    )
    def k(hidden_hbm, pos_hbm, xs_hbm, idx_v, rows_v, sem):
        wid = lax.axis_index("s") * SC_NC + lax.axis_index("c")
        base = wid * TPW
        pltpu.sync_copy(pos_hbm.at[pl.ds(base, TPW)], idx_v)
        pltpu.sync_copy(hidden_hbm.at[pl.ds(base, TPW), :], rows_v)
        pltpu.async_copy(rows_v, xs_hbm.at[idx_v], sem).wait()

    return k(hidden, pos)


def _mlp_body(be_ref, xs_ref, wg_ref, wu_ref, wd_ref, ys_ref):
    i = pl.program_id(0)

    @pl.when(be_ref[i] < E)
    def _compute():
        x = xs_ref[...]
        g = jnp.dot(x, wg_ref[0], preferred_element_type=jnp.float32)
        u = jnp.dot(x, wu_ref[0], preferred_element_type=jnp.float32)
        h = g * jax.nn.sigmoid(g) * u
        ys_ref[...] = jnp.dot(h, wd_ref[0],
                              preferred_element_type=jnp.float32)


def _mlp_call(be, xs, w_gate, w_up, w_down):
    grid_spec = pltpu.PrefetchScalarGridSpec(
        num_scalar_prefetch=1,
        grid=(NB,),
        in_specs=[
            pl.BlockSpec((BLK, D), lambda i, be: (i, 0)),
            pl.BlockSpec((1, D, FF),
                         lambda i, be: (jnp.minimum(be[i], E - 1), 0, 0)),
            pl.BlockSpec((1, D, FF),
                         lambda i, be: (jnp.minimum(be[i], E - 1), 0, 0)),
            pl.BlockSpec((1, FF, D),
                         lambda i, be: (jnp.minimum(be[i], E - 1), 0, 0)),
        ],
        out_specs=pl.BlockSpec((BLK, D), lambda i, be: (i, 0)),
    )
    return pl.pallas_call(
        _mlp_body,
        grid_spec=grid_spec,
        out_shape=jax.ShapeDtypeStruct((P, D), jnp.float32),
    )(be, xs, w_gate, w_up, w_down)


def _sc_combine(ys, pos):
    mesh = plsc.VectorSubcoreMesh(core_axis_name="c", subcore_axis_name="s",
                                  num_cores=SC_NC, num_subcores=SC_NS)

    @functools.partial(
        pl.kernel,
        out_type=jax.ShapeDtypeStruct((T, D), jnp.float32),
        mesh=mesh,
        scratch_types=[
            pltpu.VMEM((TPW,), jnp.int32),
            pltpu.VMEM((TPW, D), jnp.float32),
            pltpu.SemaphoreType.DMA,
        ],
    )
    def k(ys_hbm, pos_hbm, yso_hbm, idx_v, rows_v, sem):
        wid = lax.axis_index("s") * SC_NC + lax.axis_index("c")
        base = wid * TPW
        pltpu.sync_copy(pos_hbm.at[pl.ds(base, TPW)], idx_v)
        pltpu.async_copy(ys_hbm.at[idx_v], rows_v, sem).wait()
        pltpu.sync_copy(rows_v, yso_hbm.at[pl.ds(base, TPW), :])

    return k(ys, pos)


BT_SH = 256
NSB = T // BT_SH


def _shared_body(x_ref, wsg_ref, wsu_ref, wsd_ref, yso_ref, wb_ref, out_ref):
    x = x_ref[...]
    g = jnp.dot(x, wsg_ref[...], preferred_element_type=jnp.float32)
    u = jnp.dot(x, wsu_ref[...], preferred_element_type=jnp.float32)
    h = g * jax.nn.sigmoid(g) * u
    y = jnp.dot(h, wsd_ref[...], preferred_element_type=jnp.float32)
    w = wb_ref[...][:, :1]
    out_ref[...] = y + w * yso_ref[...]


def _shared_call(hidden, ws_gate, ws_up, ws_down, yso, wb):
    return pl.pallas_call(
        _shared_body,
        grid=(NSB,),
        in_specs=[
            pl.BlockSpec((BT_SH, D), lambda i: (i, 0)),
            pl.BlockSpec((D, FF), lambda i: (0, 0)),
            pl.BlockSpec((D, FF), lambda i: (0, 0)),
            pl.BlockSpec((FF, D), lambda i: (0, 0)),
            pl.BlockSpec((BT_SH, D), lambda i: (i, 0)),
            pl.BlockSpec((BT_SH, LANES), lambda i: (i, 0)),
        ],
        out_specs=pl.BlockSpec((BT_SH, D), lambda i: (i, 0)),
        out_shape=jax.ShapeDtypeStruct((T, D), jnp.float32),
    )(hidden, ws_gate, ws_up, ws_down, yso, wb)


def kernel(hidden_states, W_router, w_gate, w_up, w_down,
           ws_gate, ws_up, ws_down):
    wr_pad = jnp.pad(W_router, ((0, 0), (0, LANES - E)))
    posb, wb, beb = _route_call(hidden_states, wr_pad)
    pos = posb[:, 0]
    be = beb[:NB, 0]
    xs = _sc_dispatch(hidden_states, pos)
    ys = _mlp_call(be, xs, w_gate, w_up, w_down)
    yso = _sc_combine(ys, pos)
    return _shared_call(hidden_states, ws_gate, ws_up, ws_down, yso, wb)


# speedup vs baseline: 2.3348x; 1.0703x over previous
import functools

import jax
import jax.numpy as jnp
from jax import lax
from jax.experimental import pallas as pl
from jax.experimental.pallas import tpu as pltpu
from jax.experimental.pallas import tpu_sc as plsc

E = 8
D = 1024
FF = 2048
T = 2048
BT = 512
NTB = T // BT
BLK = 128
P = 3072
NB = P // BLK
LANES = 128

SC_NC = 2
SC_NS = 16
SC_NW = SC_NC * SC_NS
TPW = T // SC_NW


def _route_body(x_ref, wr_ref, pos_ref, wb_ref, beb_ref,
                rank_s, eid_s, wgt_s, cnt_s):
    i = pl.program_id(0)

    @pl.when(i < NTB)
    def _phase1():
        @pl.when(i == 0)
        def _init():
            cnt_s[...] = jnp.zeros_like(cnt_s)

        x = x_ref[...]
        wr = wr_ref[...]
        logits = jnp.dot(x, wr, preferred_element_type=jnp.float32)
        lane = lax.broadcasted_iota(jnp.int32, (BT, LANES), 1)
        valid = lane < E
        neg = jnp.where(valid, logits, -1e30)
        m = jnp.max(neg, axis=1, keepdims=True)
        eid = jnp.min(jnp.where((neg == m) & valid, lane, LANES),
                      axis=1, keepdims=True)
        oh = (lane == eid).astype(jnp.float32)
        r_io = lax.broadcasted_iota(jnp.int32, (BT, BT), 0)
        c_io = lax.broadcasted_iota(jnp.int32, (BT, BT), 1)
        tri = (r_io > c_io).astype(jnp.float32)
        rank_loc = jnp.dot(tri, oh, preferred_element_type=jnp.float32)
        rank = rank_loc + cnt_s[...]
        w = jax.nn.sigmoid(m)
        rank_s[pl.ds(i * BT, BT), :] = jnp.sum(rank * oh, axis=1, keepdims=True)
        eid_s[pl.ds(i * BT, BT), :] = eid
        wgt_s[pl.ds(i * BT, BT), :] = w
        cnt_s[...] = cnt_s[...] + jnp.sum(oh, axis=0, keepdims=True)

    @pl.when(i >= NTB)
    def _phase2():
        j = i - NTB
        cnt = cnt_s[...]
        cnt_pad = jnp.ceil(cnt / BLK) * BLK
        r_io = lax.broadcasted_iota(jnp.int32, (LANES, LANES), 0)
        c_io = lax.broadcasted_iota(jnp.int32, (LANES, LANES), 1)
        tri_u = (r_io < c_io).astype(jnp.float32)
        pad_off = jnp.dot(cnt_pad, tri_u, preferred_element_type=jnp.float32)
        pad_end = pad_off + cnt_pad

        eid = eid_s[pl.ds(j * BT, BT), :]
        rank = rank_s[pl.ds(j * BT, BT), :]
        lane = lax.broadcasted_iota(jnp.int32, (BT, LANES), 1)
        oh = (lane == eid).astype(jnp.float32)
        off_tok = jnp.sum(oh * pad_off, axis=1, keepdims=True)
        pos = (off_tok + rank).astype(jnp.int32)
        pos_ref[...] = jnp.broadcast_to(pos, (BT, LANES))
        wb_ref[...] = jnp.broadcast_to(wgt_s[pl.ds(j * BT, BT), :], (BT, LANES))

        @pl.when(j == 0)
        def _meta():
            blk_start = (lax.broadcasted_iota(jnp.int32, (LANES, LANES), 0)
                         * BLK).astype(jnp.float32)
            lane2 = lax.broadcasted_iota(jnp.int32, (LANES, LANES), 1)
            endb = jnp.broadcast_to(pad_end, (LANES, LANES))
            done = ((blk_start >= endb) & (lane2 < E)).astype(jnp.float32)
            be = jnp.sum(done, axis=1, keepdims=True)
            beb_ref[...] = jnp.broadcast_to(be.astype(jnp.int32),
                                            (LANES, LANES))


def _route_call(hidden, wr_pad):
    return pl.pallas_call(
        _route_body,
        grid=(2 * NTB,),
        in_specs=[
            pl.BlockSpec((BT, D), lambda i: (jnp.where(i < NTB, i, 0), 0)),
            pl.BlockSpec((D, LANES), lambda i: (0, 0)),
        ],
        out_specs=[
            pl.BlockSpec((BT, LANES),
                         lambda i: (jnp.where(i < NTB, 0, i - NTB), 0)),
            pl.BlockSpec((BT, LANES),
                         lambda i: (jnp.where(i < NTB, 0, i - NTB), 0)),
            pl.BlockSpec((LANES, LANES), lambda i: (0, 0)),
        ],
        out_shape=[
            jax.ShapeDtypeStruct((T, LANES), jnp.int32),
            jax.ShapeDtypeStruct((T, LANES), jnp.float32),
            jax.ShapeDtypeStruct((LANES, LANES), jnp.int32),
        ],
        scratch_shapes=[
            pltpu.VMEM((T, 1), jnp.float32),
            pltpu.VMEM((T, 1), jnp.int32),
            pltpu.VMEM((T, 1), jnp.float32),
            pltpu.VMEM((1, LANES), jnp.float32),
        ],
    )(hidden, wr_pad)


def _sc_dispatch(hidden, pos):
    mesh = plsc.VectorSubcoreMesh(core_axis_name="c", subcore_axis_name="s",
                                  num_cores=SC_NC, num_subcores=SC_NS)

    @functools.partial(
        pl.kernel,
        out_type=jax.ShapeDtypeStruct((P, D), jnp.float32),
        mesh=mesh,
        scratch_types=[
            pltpu.VMEM((TPW,), jnp.int32),
            pltpu.VMEM((TPW, D), jnp.float32),
            pltpu.SemaphoreType.DMA,
        ],
    )
    def k(hidden_hbm, pos_hbm, xs_hbm, idx_v, rows_v, sem):
        wid = lax.axis_index("s") * SC_NC + lax.axis_index("c")
        base = wid * TPW
        pltpu.sync_copy(pos_hbm.at[pl.ds(base, TPW)], idx_v)
        pltpu.sync_copy(hidden_hbm.at[pl.ds(base, TPW), :], rows_v)
        pltpu.async_copy(rows_v, xs_hbm.at[idx_v], sem).wait()

    return k(hidden, pos)


def _mlp_body(be_ref, xs_ref, wg_ref, wu_ref, wd_ref, ys_ref):
    i = pl.program_id(0)

    @pl.when(be_ref[i] < E)
    def _compute():
        x = xs_ref[...]
        g = jnp.dot(x, wg_ref[0], preferred_element_type=jnp.float32)
        u = jnp.dot(x, wu_ref[0], preferred_element_type=jnp.float32)
        h = g * jax.nn.sigmoid(g) * u
        ys_ref[...] = jnp.dot(h, wd_ref[0],
                              preferred_element_type=jnp.float32)


def _mlp_call(be, xs, w_gate, w_up, w_down):
    grid_spec = pltpu.PrefetchScalarGridSpec(
        num_scalar_prefetch=1,
        grid=(NB,),
        in_specs=[
            pl.BlockSpec((BLK, D), lambda i, be: (i, 0)),
            pl.BlockSpec((1, D, FF),
                         lambda i, be: (jnp.minimum(be[i], E - 1), 0, 0)),
            pl.BlockSpec((1, D, FF),
                         lambda i, be: (jnp.minimum(be[i], E - 1), 0, 0)),
            pl.BlockSpec((1, FF, D),
                         lambda i, be: (jnp.minimum(be[i], E - 1), 0, 0)),
        ],
        out_specs=pl.BlockSpec((BLK, D), lambda i, be: (i, 0)),
    )
    return pl.pallas_call(
        _mlp_body,
        grid_spec=grid_spec,
        out_shape=jax.ShapeDtypeStruct((P, D), jnp.float32),
    )(be, xs, w_gate, w_up, w_down)


def _sc_combine(ys, pos):
    mesh = plsc.VectorSubcoreMesh(core_axis_name="c", subcore_axis_name="s",
                                  num_cores=SC_NC, num_subcores=SC_NS)

    @functools.partial(
        pl.kernel,
        out_type=jax.ShapeDtypeStruct((T, D), jnp.float32),
        mesh=mesh,
        scratch_types=[
            pltpu.VMEM((TPW,), jnp.int32),
            pltpu.VMEM((TPW, D), jnp.float32),
            pltpu.SemaphoreType.DMA,
        ],
    )
    def k(ys_hbm, pos_hbm, yso_hbm, idx_v, rows_v, sem):
        wid = lax.axis_index("s") * SC_NC + lax.axis_index("c")
        base = wid * TPW
        pltpu.sync_copy(pos_hbm.at[pl.ds(base, TPW)], idx_v)
        pltpu.async_copy(ys_hbm.at[idx_v], rows_v, sem).wait()
        pltpu.sync_copy(rows_v, yso_hbm.at[pl.ds(base, TPW), :])

    return k(ys, pos)


BT_SH = 256
NSB = T // BT_SH


def _shared_body(x_ref, wsg_ref, wsu_ref, wsd_ref, yso_ref, wb_ref, out_ref):
    x = x_ref[...]
    g = jnp.dot(x, wsg_ref[...], preferred_element_type=jnp.float32)
    u = jnp.dot(x, wsu_ref[...], preferred_element_type=jnp.float32)
    h = g * jax.nn.sigmoid(g) * u
    y = jnp.dot(h, wsd_ref[...], preferred_element_type=jnp.float32)
    w = wb_ref[...][:, :1]
    out_ref[...] = y + w * yso_ref[...]


def _shared_call(hidden, ws_gate, ws_up, ws_down, yso, wb):
    return pl.pallas_call(
        _shared_body,
        grid=(NSB,),
        in_specs=[
            pl.BlockSpec((BT_SH, D), lambda i: (i, 0)),
            pl.BlockSpec((D, FF), lambda i: (0, 0)),
            pl.BlockSpec((D, FF), lambda i: (0, 0)),
            pl.BlockSpec((FF, D), lambda i: (0, 0)),
            pl.BlockSpec((BT_SH, D), lambda i: (i, 0)),
            pl.BlockSpec((BT_SH, LANES), lambda i: (i, 0)),
        ],
        out_specs=pl.BlockSpec((BT_SH, D), lambda i: (i, 0)),
        out_shape=jax.ShapeDtypeStruct((T, D), jnp.float32),
    )(hidden, ws_gate, ws_up, ws_down, yso, wb)


def kernel(hidden_states, W_router, w_gate, w_up, w_down,
           ws_gate, ws_up, ws_down):
    wr_pad = jnp.pad(W_router, ((0, 0), (0, LANES - E)))
    posb, wb, beb = _route_call(hidden_states, wr_pad)
    pos = posb[:, 0]
    be = beb[:NB, 0]
    xs = _sc_dispatch(hidden_states, pos)
    ys = _mlp_call(be, xs, w_gate, w_up, w_down)
    yso = _sc_combine(ys, pos)
    return _shared_call(hidden_states, ws_gate, ws_up, ws_down, yso, wb)
